# Initial kernel scaffold; baseline (speedup 1.0000x reference)
import jax
import jax.numpy as jnp
from jax.experimental import pallas as pl


def kernel(queries, keys, values, gate_cmp, gate_slc, gate_swa):
    raise NotImplementedError("write your pallas kernel here")


# speedup vs baseline: 26.6513x; 26.6513x over previous
import functools

import jax
import jax.numpy as jnp
from jax.experimental import pallas as pl
from jax.experimental.pallas import tpu as pltpu

BS = 64
NB = 128
K_TOP = 16
TQ = 256
CB = 16
NEG = -1e30


def _compress_kernel(k_ref, v_ref, ck_ref, cv_ref):
    k = k_ref[0]
    v = v_ref[0]
    d = k.shape[-1]
    ck_ref[0] = jnp.mean(k.reshape(CB, BS, d), axis=1)
    cv_ref[0] = jnp.mean(v.reshape(CB, BS, d), axis=1)


def _nsa_kernel(q_ref, ck_ref, cv_ref, gc_ref, gs_ref, o_ref, *, scale):
    q = q_ref[0]
    ck = ck_ref[0]
    cv = cv_ref[0]
    gc = gc_ref[0]
    gs = gs_ref[0]

    s = jax.lax.dot_general(q, ck, (((1,), (1,)), ((), ())),
                            preferred_element_type=jnp.float32) * scale

    m1 = jnp.max(s, axis=1, keepdims=True)
    e1 = jnp.exp(s - m1)
    p1 = e1 / jnp.sum(e1, axis=1, keepdims=True)

    t0 = pl.program_id(1) * TQ
    rows = jax.lax.broadcasted_iota(jnp.int32, (TQ, NB), 0) + t0
    cols = jax.lax.broadcasted_iota(jnp.int32, (TQ, NB), 1)
    allowed = cols <= rows // BS
    sm = jnp.where(allowed, s, NEG)

    m2 = jnp.max(sm, axis=1, keepdims=True)
    e2 = jnp.exp(sm - m2)
    p2 = e2 / jnp.sum(e2, axis=1, keepdims=True)

    v = sm
    for _ in range(K_TOP - 1):
        v = jnp.where(v >= jnp.max(v, axis=1, keepdims=True), NEG, v)
    thresh = jnp.max(v, axis=1, keepdims=True)
    p2k = jnp.where(sm >= thresh, p2, 0.0)

    w = gc * p1 + gs * p2k
    o_ref[0] = jax.lax.dot_general(w, cv, (((1,), (0,)), ((), ())),
                                   preferred_element_type=jnp.float32)


@jax.jit
def kernel(queries, keys, values, gate_cmp, gate_slc, gate_swa):
    B, T, D = queries.shape
    scale = D ** (-0.5)
    n_ctiles = NB // CB

    comp_k, comp_v = pl.pallas_call(
        _compress_kernel,
        grid=(B, n_ctiles),
        in_specs=[
            pl.BlockSpec((1, CB * BS, D), lambda b, c: (b, c, 0)),
            pl.BlockSpec((1, CB * BS, D), lambda b, c: (b, c, 0)),
        ],
        out_specs=[
            pl.BlockSpec((1, CB, D), lambda b, c: (b, c, 0)),
            pl.BlockSpec((1, CB, D), lambda b, c: (b, c, 0)),
        ],
        out_shape=[
            jax.ShapeDtypeStruct((B, NB, D), jnp.float32),
            jax.ShapeDtypeStruct((B, NB, D), jnp.float32),
        ],
        compiler_params=pltpu.CompilerParams(
            dimension_semantics=("parallel", "parallel")),
    )(keys, values)

    gc3 = gate_cmp[:, :, None]
    gs3 = gate_slc[:, :, None]

    out = pl.pallas_call(
        functools.partial(_nsa_kernel, scale=scale),
        grid=(B, T // TQ),
        in_specs=[
            pl.BlockSpec((1, TQ, D), lambda b, t: (b, t, 0)),
            pl.BlockSpec((1, NB, D), lambda b, t: (b, 0, 0)),
            pl.BlockSpec((1, NB, D), lambda b, t: (b, 0, 0)),
            pl.BlockSpec((1, TQ, 1), lambda b, t: (b, t, 0)),
            pl.BlockSpec((1, TQ, 1), lambda b, t: (b, t, 0)),
        ],
        out_specs=pl.BlockSpec((1, TQ, D), lambda b, t: (b, t, 0)),
        out_shape=jax.ShapeDtypeStruct((B, T, D), jnp.float32),
        compiler_params=pltpu.CompilerParams(
            dimension_semantics=("parallel", "arbitrary")),
    )(queries, comp_k, comp_v, gc3, gs3)

    return out


# speedup vs baseline: 37.4925x; 1.4068x over previous
import functools

import jax
import jax.numpy as jnp
from jax.experimental import pallas as pl
from jax.experimental.pallas import tpu as pltpu

BS = 64
NB = 128
K_TOP = 16
TQ = 256
CB = 16
NEG = -1e30


def _compress_kernel(k_ref, v_ref, ck_ref, cv_ref):
    k = k_ref[0]
    v = v_ref[0]
    d = k.shape[-1]
    ck_ref[0] = jnp.mean(k.reshape(CB, BS, d), axis=1)
    cv_ref[0] = jnp.mean(v.reshape(CB, BS, d), axis=1)


def _nsa_kernel(q_ref, ck_ref, cv_ref, gc_ref, gs_ref, o_ref, *, scale):
    q = q_ref[0]
    ck = ck_ref[0]
    cv = cv_ref[0]
    gc = gc_ref[0, 0]
    gs = gs_ref[0, 0]

    st = jax.lax.dot_general(ck, q, (((1,), (1,)), ((), ())),
                             preferred_element_type=jnp.float32) * scale

    m1 = jnp.max(st, axis=0, keepdims=True)
    e1 = jnp.exp(st - m1)
    s1 = jnp.sum(e1, axis=0, keepdims=True)

    t0 = pl.program_id(1) * TQ
    nids = jax.lax.broadcasted_iota(jnp.int32, (NB, TQ), 0)
    tids = jax.lax.broadcasted_iota(jnp.int32, (NB, TQ), 1) + t0
    allowed = nids <= tids // BS
    e2 = jnp.where(allowed, e1, 0.0)
    s2 = jnp.sum(e2, axis=0, keepdims=True)

    v = e2
    for _ in range(K_TOP - 1):
        v = jnp.where(v >= jnp.max(v, axis=0, keepdims=True), -1.0, v)
    thresh = jnp.max(v, axis=0, keepdims=True)
    e2k = jnp.where(e2 >= thresh, e2, 0.0)

    w = e1 * (gc / s1) + e2k * (gs / s2)
    o_ref[0] = jax.lax.dot_general(w, cv, (((0,), (0,)), ((), ())),
                                   preferred_element_type=jnp.float32)


@jax.jit
def kernel(queries, keys, values, gate_cmp, gate_slc, gate_swa):
    B, T, D = queries.shape
    scale = D ** (-0.5)
    n_ctiles = NB // CB

    comp_k, comp_v = pl.pallas_call(
        _compress_kernel,
        grid=(B, n_ctiles),
        in_specs=[
            pl.BlockSpec((1, CB * BS, D), lambda b, c: (b, c, 0)),
            pl.BlockSpec((1, CB * BS, D), lambda b, c: (b, c, 0)),
        ],
        out_specs=[
            pl.BlockSpec((1, CB, D), lambda b, c: (b, c, 0)),
            pl.BlockSpec((1, CB, D), lambda b, c: (b, c, 0)),
        ],
        out_shape=[
            jax.ShapeDtypeStruct((B, NB, D), jnp.float32),
            jax.ShapeDtypeStruct((B, NB, D), jnp.float32),
        ],
        compiler_params=pltpu.CompilerParams(
            dimension_semantics=("parallel", "parallel")),
    )(keys, values)

    gc3 = gate_cmp.reshape(B, T // TQ, 1, TQ)
    gs3 = gate_slc.reshape(B, T // TQ, 1, TQ)

    out = pl.pallas_call(
        functools.partial(_nsa_kernel, scale=scale),
        grid=(B, T // TQ),
        in_specs=[
            pl.BlockSpec((1, TQ, D), lambda b, t: (b, t, 0)),
            pl.BlockSpec((1, NB, D), lambda b, t: (b, 0, 0)),
            pl.BlockSpec((1, NB, D), lambda b, t: (b, 0, 0)),
            pl.BlockSpec((1, 1, 1, TQ), lambda b, t: (b, t, 0, 0)),
            pl.BlockSpec((1, 1, 1, TQ), lambda b, t: (b, t, 0, 0)),
        ],
        out_specs=pl.BlockSpec((1, TQ, D), lambda b, t: (b, t, 0)),
        out_shape=jax.ShapeDtypeStruct((B, T, D), jnp.float32),
        compiler_params=pltpu.CompilerParams(
            dimension_semantics=("parallel", "arbitrary")),
    )(queries, comp_k, comp_v, gc3, gs3)

    return out


# speedup vs baseline: 48.3147x; 1.2887x over previous
import functools

import jax
import jax.numpy as jnp
from jax.experimental import pallas as pl
from jax.experimental.pallas import tpu as pltpu

BS = 64
NB = 128
K_TOP = 16
TQ = 512
CB = 16
NEG = -1e30


def _compress_kernel(k_ref, v_ref, ck_ref, cv_ref):
    k = k_ref[0]
    v = v_ref[0]
    d = k.shape[-1]
    ck_ref[0] = jnp.mean(k.reshape(CB, BS, d), axis=1)
    cv_ref[0] = jnp.mean(v.reshape(CB, BS, d), axis=1)


def _nsa_kernel(q_ref, ck_ref, cv_ref, gc_ref, gs_ref, o_ref, *, scale):
    q = q_ref[0]
    ck = ck_ref[0]
    cv = cv_ref[0]
    gc = gc_ref[0, 0]
    gs = gs_ref[0, 0]

    st = jax.lax.dot_general(ck, q, (((1,), (1,)), ((), ())),
                             preferred_element_type=jnp.float32) * scale

    m1 = jnp.max(st, axis=0, keepdims=True)
    e1 = jnp.exp(st - m1)
    s1 = jnp.sum(e1, axis=0, keepdims=True)

    t0 = pl.program_id(1) * TQ
    nids = jax.lax.broadcasted_iota(jnp.int32, (NB, TQ), 0)
    tids = jax.lax.broadcasted_iota(jnp.int32, (NB, TQ), 1) + t0
    allowed = nids <= tids // BS
    e2 = jnp.where(allowed, e1, 0.0)
    s2 = jnp.sum(e2, axis=0, keepdims=True)

    v = e2
    for _ in range(K_TOP - 1):
        v = jnp.where(v >= jnp.max(v, axis=0, keepdims=True), -1.0, v)
    thresh = jnp.max(v, axis=0, keepdims=True)
    e2k = jnp.where(e2 >= thresh, e2, 0.0)

    w = e1 * (gc / s1) + e2k * (gs / s2)
    o_ref[0] = jax.lax.dot_general(w, cv, (((0,), (0,)), ((), ())),
                                   preferred_element_type=jnp.float32)


@jax.jit
def kernel(queries, keys, values, gate_cmp, gate_slc, gate_swa):
    B, T, D = queries.shape
    scale = D ** (-0.5)
    n_ctiles = NB // CB

    comp_k, comp_v = pl.pallas_call(
        _compress_kernel,
        grid=(B, n_ctiles),
        in_specs=[
            pl.BlockSpec((1, CB * BS, D), lambda b, c: (b, c, 0)),
            pl.BlockSpec((1, CB * BS, D), lambda b, c: (b, c, 0)),
        ],
        out_specs=[
            pl.BlockSpec((1, CB, D), lambda b, c: (b, c, 0)),
            pl.BlockSpec((1, CB, D), lambda b, c: (b, c, 0)),
        ],
        out_shape=[
            jax.ShapeDtypeStruct((B, NB, D), jnp.float32),
            jax.ShapeDtypeStruct((B, NB, D), jnp.float32),
        ],
        compiler_params=pltpu.CompilerParams(
            dimension_semantics=("parallel", "parallel")),
    )(keys, values)

    gc3 = gate_cmp.reshape(B, T // TQ, 1, TQ)
    gs3 = gate_slc.reshape(B, T // TQ, 1, TQ)

    out = pl.pallas_call(
        functools.partial(_nsa_kernel, scale=scale),
        grid=(B, T // TQ),
        in_specs=[
            pl.BlockSpec((1, TQ, D), lambda b, t: (b, t, 0)),
            pl.BlockSpec((1, NB, D), lambda b, t: (b, 0, 0)),
            pl.BlockSpec((1, NB, D), lambda b, t: (b, 0, 0)),
            pl.BlockSpec((1, 1, 1, TQ), lambda b, t: (b, t, 0, 0)),
            pl.BlockSpec((1, 1, 1, TQ), lambda b, t: (b, t, 0, 0)),
        ],
        out_specs=pl.BlockSpec((1, TQ, D), lambda b, t: (b, t, 0)),
        out_shape=jax.ShapeDtypeStruct((B, T, D), jnp.float32),
        compiler_params=pltpu.CompilerParams(
            dimension_semantics=("parallel", "arbitrary")),
    )(queries, comp_k, comp_v, gc3, gs3)

    return out


# speedup vs baseline: 55.4962x; 1.1486x over previous
import functools

import jax
import jax.numpy as jnp
from jax.experimental import pallas as pl
from jax.experimental.pallas import tpu as pltpu

BS = 64
NB = 128
K_TOP = 16
TQ = 1024
CB = 16
NEG = -1e30


def _compress_kernel(k_ref, v_ref, ck_ref, cv_ref):
    k = k_ref[0]
    v = v_ref[0]
    d = k.shape[-1]
    ck_ref[0] = jnp.mean(k.reshape(CB, BS, d), axis=1)
    cv_ref[0] = jnp.mean(v.reshape(CB, BS, d), axis=1)


def _nsa_kernel(q_ref, ck_ref, cv_ref, gc_ref, gs_ref, o_ref, *, scale):
    q = q_ref[0]
    ck = ck_ref[0]
    cv = cv_ref[0]
    gc = gc_ref[0, 0]
    gs = gs_ref[0, 0]

    st = jax.lax.dot_general(ck, q, (((1,), (1,)), ((), ())),
                             preferred_element_type=jnp.float32) * scale

    m1 = jnp.max(st, axis=0, keepdims=True)
    e1 = jnp.exp(st - m1)
    s1 = jnp.sum(e1, axis=0, keepdims=True)

    t0 = pl.program_id(1) * TQ
    nids = jax.lax.broadcasted_iota(jnp.int32, (NB, TQ), 0)
    tids = jax.lax.broadcasted_iota(jnp.int32, (NB, TQ), 1) + t0
    allowed = nids <= tids // BS
    e2 = jnp.where(allowed, e1, 0.0)
    s2 = jnp.sum(e2, axis=0, keepdims=True)

    v = e2
    for _ in range(K_TOP - 1):
        v = jnp.where(v >= jnp.max(v, axis=0, keepdims=True), -1.0, v)
    thresh = jnp.max(v, axis=0, keepdims=True)
    e2k = jnp.where(e2 >= thresh, e2, 0.0)

    w = e1 * (gc / s1) + e2k * (gs / s2)
    o_ref[0] = jax.lax.dot_general(w, cv, (((0,), (0,)), ((), ())),
                                   preferred_element_type=jnp.float32)


@jax.jit
def kernel(queries, keys, values, gate_cmp, gate_slc, gate_swa):
    B, T, D = queries.shape
    scale = D ** (-0.5)
    n_ctiles = NB // CB

    comp_k, comp_v = pl.pallas_call(
        _compress_kernel,
        grid=(B, n_ctiles),
        in_specs=[
            pl.BlockSpec((1, CB * BS, D), lambda b, c: (b, c, 0)),
            pl.BlockSpec((1, CB * BS, D), lambda b, c: (b, c, 0)),
        ],
        out_specs=[
            pl.BlockSpec((1, CB, D), lambda b, c: (b, c, 0)),
            pl.BlockSpec((1, CB, D), lambda b, c: (b, c, 0)),
        ],
        out_shape=[
            jax.ShapeDtypeStruct((B, NB, D), jnp.float32),
            jax.ShapeDtypeStruct((B, NB, D), jnp.float32),
        ],
        compiler_params=pltpu.CompilerParams(
            dimension_semantics=("parallel", "parallel")),
    )(keys, values)

    gc3 = gate_cmp.reshape(B, T // TQ, 1, TQ)
    gs3 = gate_slc.reshape(B, T // TQ, 1, TQ)

    out = pl.pallas_call(
        functools.partial(_nsa_kernel, scale=scale),
        grid=(B, T // TQ),
        in_specs=[
            pl.BlockSpec((1, TQ, D), lambda b, t: (b, t, 0)),
            pl.BlockSpec((1, NB, D), lambda b, t: (b, 0, 0)),
            pl.BlockSpec((1, NB, D), lambda b, t: (b, 0, 0)),
            pl.BlockSpec((1, 1, 1, TQ), lambda b, t: (b, t, 0, 0)),
            pl.BlockSpec((1, 1, 1, TQ), lambda b, t: (b, t, 0, 0)),
        ],
        out_specs=pl.BlockSpec((1, TQ, D), lambda b, t: (b, t, 0)),
        out_shape=jax.ShapeDtypeStruct((B, T, D), jnp.float32),
        compiler_params=pltpu.CompilerParams(
            dimension_semantics=("parallel", "arbitrary")),
    )(queries, comp_k, comp_v, gc3, gs3)

    return out


# speedup vs baseline: 59.5940x; 1.0738x over previous
import functools

import jax
import jax.numpy as jnp
from jax.experimental import pallas as pl
from jax.experimental.pallas import tpu as pltpu

BS = 64
NB = 128
K_TOP = 16
TQ = 2048
CB = 16
NEG = -1e30


def _compress_kernel(k_ref, v_ref, ck_ref, cv_ref):
    k = k_ref[0]
    v = v_ref[0]
    d = k.shape[-1]
    ck_ref[0] = jnp.mean(k.reshape(CB, BS, d), axis=1)
    cv_ref[0] = jnp.mean(v.reshape(CB, BS, d), axis=1)


def _nsa_kernel(q_ref, ck_ref, cv_ref, gc_ref, gs_ref, o_ref, *, scale):
    q = q_ref[0]
    ck = ck_ref[0]
    cv = cv_ref[0]
    gc = gc_ref[0, 0]
    gs = gs_ref[0, 0]

    st = jax.lax.dot_general(ck, q, (((1,), (1,)), ((), ())),
                             preferred_element_type=jnp.float32) * scale

    m1 = jnp.max(st, axis=0, keepdims=True)
    e1 = jnp.exp(st - m1)
    s1 = jnp.sum(e1, axis=0, keepdims=True)

    t0 = pl.program_id(1) * TQ
    nids = jax.lax.broadcasted_iota(jnp.int32, (NB, TQ), 0)
    tids = jax.lax.broadcasted_iota(jnp.int32, (NB, TQ), 1) + t0
    allowed = nids <= tids // BS
    e2 = jnp.where(allowed, e1, 0.0)
    s2 = jnp.sum(e2, axis=0, keepdims=True)

    v = e2
    for _ in range(K_TOP - 1):
        v = jnp.where(v >= jnp.max(v, axis=0, keepdims=True), -1.0, v)
    thresh = jnp.max(v, axis=0, keepdims=True)
    e2k = jnp.where(e2 >= thresh, e2, 0.0)

    w = e1 * (gc / s1) + e2k * (gs / s2)
    o_ref[0] = jax.lax.dot_general(w, cv, (((0,), (0,)), ((), ())),
                                   preferred_element_type=jnp.float32)


@jax.jit
def kernel(queries, keys, values, gate_cmp, gate_slc, gate_swa):
    B, T, D = queries.shape
    scale = D ** (-0.5)
    n_ctiles = NB // CB

    comp_k, comp_v = pl.pallas_call(
        _compress_kernel,
        grid=(B, n_ctiles),
        in_specs=[
            pl.BlockSpec((1, CB * BS, D), lambda b, c: (b, c, 0)),
            pl.BlockSpec((1, CB * BS, D), lambda b, c: (b, c, 0)),
        ],
        out_specs=[
            pl.BlockSpec((1, CB, D), lambda b, c: (b, c, 0)),
            pl.BlockSpec((1, CB, D), lambda b, c: (b, c, 0)),
        ],
        out_shape=[
            jax.ShapeDtypeStruct((B, NB, D), jnp.float32),
            jax.ShapeDtypeStruct((B, NB, D), jnp.float32),
        ],
        compiler_params=pltpu.CompilerParams(
            dimension_semantics=("parallel", "parallel")),
    )(keys, values)

    gc3 = gate_cmp.reshape(B, T // TQ, 1, TQ)
    gs3 = gate_slc.reshape(B, T // TQ, 1, TQ)

    out = pl.pallas_call(
        functools.partial(_nsa_kernel, scale=scale),
        grid=(B, T // TQ),
        in_specs=[
            pl.BlockSpec((1, TQ, D), lambda b, t: (b, t, 0)),
            pl.BlockSpec((1, NB, D), lambda b, t: (b, 0, 0)),
            pl.BlockSpec((1, NB, D), lambda b, t: (b, 0, 0)),
            pl.BlockSpec((1, 1, 1, TQ), lambda b, t: (b, t, 0, 0)),
            pl.BlockSpec((1, 1, 1, TQ), lambda b, t: (b, t, 0, 0)),
        ],
        out_specs=pl.BlockSpec((1, TQ, D), lambda b, t: (b, t, 0)),
        out_shape=jax.ShapeDtypeStruct((B, T, D), jnp.float32),
        compiler_params=pltpu.CompilerParams(
            dimension_semantics=("parallel", "arbitrary")),
    )(queries, comp_k, comp_v, gc3, gs3)

    return out
